# baseline (speedup 1.0000x reference)
import functools
import math

import jax
import jax.numpy as jnp
from jax import lax
from jax.experimental import pallas as pl
from jax.experimental.pallas import tpu as pltpu
from jax.experimental.pallas import tpu_sc as plsc

_N = 2048
_NCLASSES = 32000
_PADDING_IDX = 0
_SMOOTHING = 0.1
_CONFIDENCE = 1.0 - _SMOOTHING
_EPS = _SMOOTHING / (_NCLASSES - 2)
_C_ROW = (_NCLASSES - 1) * _EPS * math.log(_EPS) + _CONFIDENCE * math.log(_CONFIDENCE)

_BR = 256
_BC = 1280
_NSTREAMS = 5


def _sc_gather_partials(ty_true, flat_prob):
    info = plsc.get_sparse_core_info()
    nc, ns, L = info.num_cores, info.num_subcores, info.num_lanes
    nw = nc * ns
    bpw = _N // nw

    mesh = plsc.VectorSubcoreMesh(core_axis_name="c", subcore_axis_name="s")

    @functools.partial(
        pl.kernel,
        mesh=mesh,
        out_type=jax.ShapeDtypeStruct((nw, L), jnp.float32),
        scratch_types=[
            pltpu.VMEM((bpw,), jnp.int32),
            pltpu.VMEM((bpw,), jnp.int32),
            pltpu.VMEM((bpw,), jnp.float32),
            pltpu.VMEM((L,), jnp.float32),
            pltpu.SemaphoreType.DMA,
        ],
    )
    def sc_kernel(ttrue_hbm, flat_hbm, out_hbm, t_v, idx_v, val_v, acc_v, sem):
        wid = lax.axis_index("c") * ns + lax.axis_index("s")
        base = wid * bpw
        pltpu.sync_copy(ttrue_hbm.at[pl.ds(base, bpw)], t_v)
        for c in range(bpw // L):
            t16 = t_v[pl.ds(c * L, L)]
            rows = base + c * L + lax.iota(jnp.int32, L)
            idx_v[pl.ds(c * L, L)] = rows * _NCLASSES + t16
        pltpu.async_copy(flat_hbm.at[idx_v], val_v, sem).wait()
        acc = jnp.zeros((L,), jnp.float32)
        for c in range(bpw // L):
            t16 = t_v[pl.ds(c * L, L)]
            v16 = val_v[pl.ds(c * L, L)]
            acc = acc + jnp.where(t16 != _PADDING_IDX, v16, 0.0)
        acc_v[...] = acc
        pltpu.sync_copy(acc_v, out_hbm.at[wid])

    return sc_kernel(ty_true, flat_prob)


def _tc_body(nr, nc, *refs):
    (*prob_refs, ttrue_ref, scpart_ref, out_ref, acc_ref) = refs
    i = pl.program_id(0)
    j = pl.program_id(1)

    @pl.when((i == 0) & (j == 0))
    def _init():
        acc_ref[0] = 0.0
        acc_ref[1] = 0.0

    t = ttrue_ref[...]
    rowsum = prob_refs[0][...].sum(axis=1, keepdims=True)
    for r in prob_refs[1:]:
        rowsum += r[...].sum(axis=1, keepdims=True)
    nonpad = t != _PADDING_IDX
    acc_ref[0] += jnp.sum(jnp.where(nonpad, rowsum, 0.0))

    @pl.when(j == 0)
    def _count():
        acc_ref[1] += jnp.sum(jnp.where(nonpad, 1.0, 0.0))

    @pl.when((i == nr - 1) & (j == nc - 1))
    def _finish():
        s2 = jnp.sum(scpart_ref[...])
        out_ref[0, 0] = (_C_ROW * acc_ref[1]
                         - _EPS * acc_ref[0]
                         - (_CONFIDENCE - _EPS) * s2)


def kernel(ty_prob, ty_true):
    sc_part = _sc_gather_partials(ty_true, ty_prob.reshape(-1))
    sc_part = sc_part.reshape(4, 128)

    nr = _N // _BR
    nc = _NCLASSES // (_BC * _NSTREAMS)
    prob_specs = [
        pl.BlockSpec((_BR, _BC), functools.partial(
            lambda k, i, j: (i, k * nc + j), k))
        for k in range(_NSTREAMS)
    ]
    out = pl.pallas_call(
        functools.partial(_tc_body, nr, nc),
        grid=(nr, nc),
        in_specs=prob_specs + [
            pl.BlockSpec((_BR, 1), lambda i, j: (i, 0)),
            pl.BlockSpec((4, 128), lambda i, j: (0, 0)),
        ],
        out_specs=pl.BlockSpec(memory_space=pltpu.SMEM),
        out_shape=jax.ShapeDtypeStruct((1, 1), jnp.float32),
        scratch_shapes=[pltpu.SMEM((2,), jnp.float32)],
        compiler_params=pltpu.CompilerParams(
            dimension_semantics=("arbitrary", "arbitrary")),
    )(*([ty_prob] * _NSTREAMS), ty_true.reshape(_N, 1), sc_part)
    return out[0, 0]


# speedup vs baseline: 1.0021x; 1.0021x over previous
import functools
import math

import jax
import jax.numpy as jnp
from jax import lax
from jax.experimental import pallas as pl
from jax.experimental.pallas import tpu as pltpu
from jax.experimental.pallas import tpu_sc as plsc

_N = 2048
_NCLASSES = 32000
_PADDING_IDX = 0
_SMOOTHING = 0.1
_CONFIDENCE = 1.0 - _SMOOTHING
_EPS = _SMOOTHING / (_NCLASSES - 2)
_C_ROW = (_NCLASSES - 1) * _EPS * math.log(_EPS) + _CONFIDENCE * math.log(_CONFIDENCE)

_BR = 128
_BC = 32000
_NSTREAMS = 1


def _sc_gather_partials(ty_true, flat_prob):
    info = plsc.get_sparse_core_info()
    nc, ns, L = info.num_cores, info.num_subcores, info.num_lanes
    nw = nc * ns
    bpw = _N // nw

    mesh = plsc.VectorSubcoreMesh(core_axis_name="c", subcore_axis_name="s")

    @functools.partial(
        pl.kernel,
        mesh=mesh,
        out_type=jax.ShapeDtypeStruct((nw, L), jnp.float32),
        scratch_types=[
            pltpu.VMEM((bpw,), jnp.int32),
            pltpu.VMEM((bpw,), jnp.int32),
            pltpu.VMEM((bpw,), jnp.float32),
            pltpu.VMEM((L,), jnp.float32),
            pltpu.SemaphoreType.DMA,
        ],
    )
    def sc_kernel(ttrue_hbm, flat_hbm, out_hbm, t_v, idx_v, val_v, acc_v, sem):
        wid = lax.axis_index("c") * ns + lax.axis_index("s")
        base = wid * bpw
        pltpu.sync_copy(ttrue_hbm.at[pl.ds(base, bpw)], t_v)
        for c in range(bpw // L):
            t16 = t_v[pl.ds(c * L, L)]
            rows = base + c * L + lax.iota(jnp.int32, L)
            idx_v[pl.ds(c * L, L)] = rows * _NCLASSES + t16
        pltpu.async_copy(flat_hbm.at[idx_v], val_v, sem).wait()
        acc = jnp.zeros((L,), jnp.float32)
        for c in range(bpw // L):
            t16 = t_v[pl.ds(c * L, L)]
            v16 = val_v[pl.ds(c * L, L)]
            acc = acc + jnp.where(t16 != _PADDING_IDX, v16, 0.0)
        acc_v[...] = acc
        pltpu.sync_copy(acc_v, out_hbm.at[wid])

    return sc_kernel(ty_true, flat_prob)


def _tc_body(nr, nc, *refs):
    (*prob_refs, ttrue_ref, scpart_ref, out_ref, acc_ref) = refs
    i = pl.program_id(0)
    j = pl.program_id(1)

    @pl.when((i == 0) & (j == 0))
    def _init():
        acc_ref[0] = 0.0
        acc_ref[1] = 0.0

    t = ttrue_ref[...]
    rowsum = prob_refs[0][...].sum(axis=1, keepdims=True)
    for r in prob_refs[1:]:
        rowsum += r[...].sum(axis=1, keepdims=True)
    nonpad = t != _PADDING_IDX
    acc_ref[0] += jnp.sum(jnp.where(nonpad, rowsum, 0.0))

    @pl.when(j == 0)
    def _count():
        acc_ref[1] += jnp.sum(jnp.where(nonpad, 1.0, 0.0))

    @pl.when((i == nr - 1) & (j == nc - 1))
    def _finish():
        s2 = jnp.sum(scpart_ref[...])
        out_ref[0, 0] = (_C_ROW * acc_ref[1]
                         - _EPS * acc_ref[0]
                         - (_CONFIDENCE - _EPS) * s2)


def kernel(ty_prob, ty_true):
    sc_part = _sc_gather_partials(ty_true, ty_prob.reshape(-1))
    sc_part = sc_part.reshape(4, 128)

    nr = _N // _BR
    nc = _NCLASSES // (_BC * _NSTREAMS)
    prob_specs = [
        pl.BlockSpec((_BR, _BC), functools.partial(
            lambda k, i, j: (i, k * nc + j), k))
        for k in range(_NSTREAMS)
    ]
    out = pl.pallas_call(
        functools.partial(_tc_body, nr, nc),
        grid=(nr, nc),
        in_specs=prob_specs + [
            pl.BlockSpec((_BR, 1), lambda i, j: (i, 0)),
            pl.BlockSpec((4, 128), lambda i, j: (0, 0)),
        ],
        out_specs=pl.BlockSpec(memory_space=pltpu.SMEM),
        out_shape=jax.ShapeDtypeStruct((1, 1), jnp.float32),
        scratch_shapes=[pltpu.SMEM((2,), jnp.float32)],
        compiler_params=pltpu.CompilerParams(
            dimension_semantics=("arbitrary", "arbitrary")),
    )(*([ty_prob] * _NSTREAMS), ty_true.reshape(_N, 1), sc_part)
    return out[0, 0]
